# baseline (speedup 1.0000x reference)
import jax
import jax.numpy as jnp
from jax.experimental import pallas as pl
from jax.experimental.pallas import tpu as pltpu

DEPTH = 4
B_TOK = 16384
CODE_DIM = 256
N_CODES = 1024
BLK = 1024
SUBS = (512, 512)


def _vq_body(z_ref, cb_ref, ct_ref, cn_ref, zq_ref, m0, m1, m2, m3):
    maps = (m0, m1, m2, m3)
    cn = cn_ref[...]
    offs = [sum(SUBS[:h]) for h in range(len(SUBS))]
    r = [z_ref[pl.ds(offs[h], SUBS[h]), :] for h in range(len(SUBS))]
    zq = [jnp.zeros((SUBS[h], CODE_DIM), jnp.float32)
          for h in range(len(SUBS))]
    for i in range(DEPTH):
        for h in range(len(SUBS)):
            SUB = SUBS[h]
            rn = jnp.sum(r[h] * r[h], axis=1, keepdims=True)
            g2 = jax.lax.dot_general(
                r[h], ct_ref[...], (((1,), (1,)), ((), ())),
                preferred_element_type=jnp.float32)
            dist = (rn + cn) - g2
            maps[i][pl.ds(offs[h], SUB), :] = dist
            idx = jnp.argmin(dist, axis=1)
            oh = (jax.lax.broadcasted_iota(jnp.int32, (SUB, N_CODES), 1)
                  == idx[:, None]).astype(jnp.float32)
            delta = jax.lax.dot_general(
                oh, cb_ref[...], (((1,), (0,)), ((), ())),
                preferred_element_type=jnp.float32)
            zq[h] = zq[h] + delta
            r[h] = r[h] - delta
    for h in range(len(SUBS)):
        zq_ref[pl.ds(offs[h], SUBS[h]), :] = zq[h]


@jax.jit
def kernel(z, codebook, codebook_t):
    cn = jnp.sum(codebook_t ** 2, axis=1)[None, :]
    ct2 = 2.0 * codebook_t
    grid = (B_TOK // BLK,)
    map_spec = pl.BlockSpec((BLK, N_CODES), lambda b: (b, 0))
    out = pl.pallas_call(
        _vq_body,
        grid=grid,
        in_specs=[
            pl.BlockSpec((BLK, CODE_DIM), lambda b: (b, 0)),
            pl.BlockSpec((N_CODES, CODE_DIM), lambda b: (0, 0)),
            pl.BlockSpec((N_CODES, CODE_DIM), lambda b: (0, 0)),
            pl.BlockSpec((1, N_CODES), lambda b: (0, 0)),
        ],
        out_specs=[
            pl.BlockSpec((BLK, CODE_DIM), lambda b: (b, 0)),
            map_spec, map_spec, map_spec, map_spec,
        ],
        out_shape=[
            jax.ShapeDtypeStruct((B_TOK, CODE_DIM), jnp.float32),
        ] + [jax.ShapeDtypeStruct((B_TOK, N_CODES), jnp.float32)] * DEPTH,
        compiler_params=pltpu.CompilerParams(
            dimension_semantics=("parallel",),
            vmem_limit_bytes=100 * 1024 * 1024),
    )(z, codebook, ct2, cn)
    return tuple(out)
